# baseline (speedup 1.0000x reference)
import functools

import jax
import jax.numpy as jnp
from jax.experimental import pallas as pl
from jax.experimental.pallas import tpu as pltpu

BN = 1024
CA = 24
NCA = 1024


def _body(x_ref, a_ref, lay_ref, k1_ref, as1_ref, an1_ref, b1_ref,
          wl_ref, bl_ref, w2_ref, as2_ref, an2_ref, b2_ref,
          out_ref,
          abf_ref, h2_ref, haug_ref, ea_ref, eb_ref, et_ref, ef_ref,
          lemb_ref, s2_ref, t2_ref,
          *, heads, chan, nc, bn, nblk):
    i = pl.program_id(0)

    def setup():
        n = x_ref.shape[0]
        xv = x_ref[...]
        ones = jnp.ones((n, 1), jnp.float32)
        zeros = jnp.zeros((n, CA - chan - 1), jnp.float32)
        pieces = []
        s1s = []
        t1s = []
        for h in range(heads):
            h1h = jnp.dot(xv, k1_ref[:, h, :],
                          preferred_element_type=jnp.float32)
            pieces += [h1h, ones, zeros]
            s1s.append(jax.lax.dot_general(
                h1h, as1_ref[h:h + 1, :], (((1,), (1,)), ((), ())),
                preferred_element_type=jnp.float32))
            t1s.append(jax.lax.dot_general(
                an1_ref[h:h + 1, :], h1h, (((1,), (1,)), ((), ())),
                preferred_element_type=jnp.float32))
        haug_ref[...] = jnp.concatenate(pieces, axis=1).astype(jnp.bfloat16)
        s1 = jnp.concatenate(s1s, axis=1)
        t1 = jnp.concatenate(t1s, axis=0)
        tmax = jnp.max(t1, axis=1, keepdims=True)
        z = s1 + tmax.T
        u1 = jnp.maximum(z, 0.2 * z)
        ea_ref[...] = jnp.exp(s1 - u1 + tmax.T)
        eb_ref[...] = jnp.exp(0.2 * s1 - u1 + 0.2 * tmax.T)
        et_ref[...] = jnp.exp(t1 - tmax)
        ef_ref[...] = jnp.exp(0.2 * (t1 - tmax))
        lemb = jnp.dot(lay_ref[...], wl_ref[...],
                       preferred_element_type=jnp.float32) + bl_ref[...][None, :]
        lemb_ref[...] = jnp.maximum(lemb, 0.0)

    def phase_a():
        j = (i - 1) % nblk
        ab = a_ref[...]
        abb = ab.astype(jnp.bfloat16)
        abf_ref[pl.ds(j * bn, bn), :] = abb
        eab = ea_ref[pl.ds(j * bn, bn), :]
        ebb = eb_ref[pl.ds(j * bn, bn), :]
        etr = et_ref[...]
        efr = ef_ref[...]
        haug = haug_ref[...]
        outs = []
        for h in range(heads):
            pz = jnp.maximum(eab[:, h:h + 1] * etr[h:h + 1, :],
                             ebb[:, h:h + 1] * efr[h:h + 1, :])
            p = pz.astype(jnp.bfloat16) * abb
            aug = jnp.dot(p, haug[:, h * CA:(h + 1) * CA],
                          preferred_element_type=jnp.float32)
            outs.append(aug[:, :chan] / aug[:, chan:chan + 1])
        x1 = jnp.concatenate(outs, axis=1) + b1_ref[...][None, :]
        x1 = jnp.where(x1 > 0, x1, jnp.exp(x1) - 1.0)
        x1g = x1 + lemb_ref[pl.ds(j * bn, bn), :]
        h2b = jnp.dot(x1g, w2_ref[:, 0, :], preferred_element_type=jnp.float32)
        ones = jnp.ones((bn, 1), jnp.float32)
        zeros = jnp.zeros((bn, NCA - nc - 1), jnp.float32)
        h2_ref[pl.ds(j * bn, bn), :] = jnp.concatenate(
            [h2b, ones, zeros], axis=1).astype(jnp.bfloat16)
        s2_ref[pl.ds(j * bn, bn), :] = jax.lax.dot_general(
            h2b, as2_ref[...], (((1,), (1,)), ((), ())),
            preferred_element_type=jnp.float32)
        t2_ref[0:1, pl.ds(j * bn, bn)] = jax.lax.dot_general(
            an2_ref[...], h2b, (((1,), (1,)), ((), ())),
            preferred_element_type=jnp.float32)

    def phase_b():
        j = i - nblk - 1
        h2f = h2_ref[...]
        s2 = s2_ref[pl.ds(j * bn, bn), :]
        t2 = t2_ref[...]
        t2max = jnp.max(t2)
        zu = s2 + t2max
        u2 = jnp.maximum(zu, 0.2 * zu)
        ea2 = jnp.exp(s2 - u2 + t2max)
        eb2 = jnp.exp(0.2 * s2 - u2 + 0.2 * t2max)
        et2 = jnp.exp(t2 - t2max)
        ef2 = jnp.exp(0.2 * (t2 - t2max))
        pz = jnp.maximum(ea2 * et2, eb2 * ef2)
        p = pz.astype(jnp.bfloat16) * abf_ref[pl.ds(j * bn, bn), :]
        agg = jnp.dot(p, h2f, preferred_element_type=jnp.float32)
        h2agg = agg[:, :nc] / agg[:, nc:nc + 1] + b2_ref[...][None, :]
        cmax = jnp.max(h2agg, axis=1, keepdims=True)
        e = jnp.exp(h2agg - cmax)
        out_ref[...] = e / jnp.sum(e, axis=1, keepdims=True)

    idx = (i > 0).astype(jnp.int32) + (i > nblk).astype(jnp.int32)
    jax.lax.switch(idx, [setup, phase_a, phase_b])


@jax.jit
def kernel(x, a, layout, kernel1, attn_s1, attn_n1, bias1, Wl, bl,
           kernel2, attn_s2, attn_n2, bias2):
    N, F = x.shape
    H, C = attn_s1.shape
    NC = attn_s2.shape[1]
    HC = H * C
    DL = layout.shape[1]
    nblk = N // BN

    def a_map(i):
        return (jnp.where(i <= nblk, jnp.maximum(i - 1, 0), i - nblk - 1), 0)

    out = pl.pallas_call(
        functools.partial(_body, heads=H, chan=C, nc=NC, bn=BN, nblk=nblk),
        grid=(1 + 2 * nblk,),
        in_specs=[
            pl.BlockSpec((N, F), lambda i: (0, 0)),
            pl.BlockSpec((BN, N), a_map),
            pl.BlockSpec((N, DL), lambda i: (0, 0)),
            pl.BlockSpec((F, H, C), lambda i: (0, 0, 0)),
            pl.BlockSpec((H, C), lambda i: (0, 0)),
            pl.BlockSpec((H, C), lambda i: (0, 0)),
            pl.BlockSpec((HC,), lambda i: (0,)),
            pl.BlockSpec((DL, HC), lambda i: (0, 0)),
            pl.BlockSpec((HC,), lambda i: (0,)),
            pl.BlockSpec((HC, 1, NC), lambda i: (0, 0, 0)),
            pl.BlockSpec((1, NC), lambda i: (0, 0)),
            pl.BlockSpec((1, NC), lambda i: (0, 0)),
            pl.BlockSpec((NC,), lambda i: (0,)),
        ],
        out_specs=pl.BlockSpec(
            (BN, NC), lambda i: (jnp.maximum(i - nblk - 1, 0), 0)),
        out_shape=jax.ShapeDtypeStruct((N, NC), jnp.float32),
        scratch_shapes=[
            pltpu.VMEM((N, N), jnp.bfloat16),
            pltpu.VMEM((N, NCA), jnp.bfloat16),
            pltpu.VMEM((N, H * CA), jnp.bfloat16),
            pltpu.VMEM((N, H), jnp.float32),
            pltpu.VMEM((N, H), jnp.float32),
            pltpu.VMEM((H, N), jnp.float32),
            pltpu.VMEM((H, N), jnp.float32),
            pltpu.VMEM((N, HC), jnp.float32),
            pltpu.VMEM((N, 1), jnp.float32),
            pltpu.VMEM((1, N), jnp.float32),
        ],
    )(x, a, layout, kernel1, attn_s1, attn_n1, bias1,
      Wl, bl, kernel2, attn_s2, attn_n2, bias2)
    return out


# speedup vs baseline: 1.0175x; 1.0175x over previous
import functools

import jax
import jax.numpy as jnp
from jax.experimental import pallas as pl
from jax.experimental.pallas import tpu as pltpu

BN = 512
CA = 24
NCA = 1024


def _body(x_ref, a_ref, lay_ref, k1_ref, as1_ref, an1_ref, b1_ref,
          wl_ref, bl_ref, w2_ref, as2_ref, an2_ref, b2_ref,
          out_ref,
          abf_ref, h2_ref, haug_ref, ea_ref, eb_ref, et_ref, ef_ref,
          lemb_ref, s2_ref, t2_ref,
          *, heads, chan, nc, bn, nblk):
    i = pl.program_id(0)

    def setup():
        n = x_ref.shape[0]
        xv = x_ref[...]
        ones = jnp.ones((n, 1), jnp.float32)
        zeros = jnp.zeros((n, CA - chan - 1), jnp.float32)
        pieces = []
        s1s = []
        t1s = []
        for h in range(heads):
            h1h = jnp.dot(xv, k1_ref[:, h, :],
                          preferred_element_type=jnp.float32)
            pieces += [h1h, ones, zeros]
            s1s.append(jax.lax.dot_general(
                h1h, as1_ref[h:h + 1, :], (((1,), (1,)), ((), ())),
                preferred_element_type=jnp.float32))
            t1s.append(jax.lax.dot_general(
                an1_ref[h:h + 1, :], h1h, (((1,), (1,)), ((), ())),
                preferred_element_type=jnp.float32))
        haug_ref[...] = jnp.concatenate(pieces, axis=1).astype(jnp.bfloat16)
        s1 = jnp.concatenate(s1s, axis=1)
        t1 = jnp.concatenate(t1s, axis=0)
        tmax = jnp.max(t1, axis=1, keepdims=True)
        z = s1 + tmax.T
        u1 = jnp.maximum(z, 0.2 * z)
        ea_ref[...] = jnp.exp(s1 - u1 + tmax.T)
        eb_ref[...] = jnp.exp(0.2 * s1 - u1 + 0.2 * tmax.T)
        et_ref[...] = jnp.exp(t1 - tmax)
        ef_ref[...] = jnp.exp(0.2 * (t1 - tmax))
        lemb = jnp.dot(lay_ref[...], wl_ref[...],
                       preferred_element_type=jnp.float32) + bl_ref[...][None, :]
        lemb_ref[...] = jnp.maximum(lemb, 0.0)

    def phase_a():
        j = (i - 1) % nblk
        ab = a_ref[...]
        abb = ab.astype(jnp.bfloat16)
        abf_ref[pl.ds(j * bn, bn), :] = abb
        eab = ea_ref[pl.ds(j * bn, bn), :]
        ebb = eb_ref[pl.ds(j * bn, bn), :]
        etr = et_ref[...]
        efr = ef_ref[...]
        haug = haug_ref[...]
        outs = []
        for h in range(heads):
            pz = jnp.maximum(eab[:, h:h + 1] * etr[h:h + 1, :],
                             ebb[:, h:h + 1] * efr[h:h + 1, :])
            p = pz.astype(jnp.bfloat16) * abb
            aug = jnp.dot(p, haug[:, h * CA:(h + 1) * CA],
                          preferred_element_type=jnp.float32)
            outs.append(aug[:, :chan] / aug[:, chan:chan + 1])
        x1 = jnp.concatenate(outs, axis=1) + b1_ref[...][None, :]
        x1 = jnp.where(x1 > 0, x1, jnp.exp(x1) - 1.0)
        x1g = x1 + lemb_ref[pl.ds(j * bn, bn), :]
        h2b = jnp.dot(x1g, w2_ref[:, 0, :], preferred_element_type=jnp.float32)
        ones = jnp.ones((bn, 1), jnp.float32)
        zeros = jnp.zeros((bn, NCA - nc - 1), jnp.float32)
        h2_ref[pl.ds(j * bn, bn), :] = jnp.concatenate(
            [h2b, ones, zeros], axis=1).astype(jnp.bfloat16)
        s2_ref[pl.ds(j * bn, bn), :] = jax.lax.dot_general(
            h2b, as2_ref[...], (((1,), (1,)), ((), ())),
            preferred_element_type=jnp.float32)
        t2_ref[0:1, pl.ds(j * bn, bn)] = jax.lax.dot_general(
            an2_ref[...], h2b, (((1,), (1,)), ((), ())),
            preferred_element_type=jnp.float32)

    def phase_b():
        j = i - nblk - 1
        h2f = h2_ref[...]
        s2 = s2_ref[pl.ds(j * bn, bn), :]
        t2 = t2_ref[...]
        t2max = jnp.max(t2)
        zu = s2 + t2max
        u2 = jnp.maximum(zu, 0.2 * zu)
        ea2 = jnp.exp(s2 - u2 + t2max)
        eb2 = jnp.exp(0.2 * s2 - u2 + 0.2 * t2max)
        et2 = jnp.exp(t2 - t2max)
        ef2 = jnp.exp(0.2 * (t2 - t2max))
        pz = jnp.maximum(ea2 * et2, eb2 * ef2)
        p = pz.astype(jnp.bfloat16) * abf_ref[pl.ds(j * bn, bn), :]
        agg = jnp.dot(p, h2f, preferred_element_type=jnp.float32)
        h2agg = agg[:, :nc] / agg[:, nc:nc + 1] + b2_ref[...][None, :]
        cmax = jnp.max(h2agg, axis=1, keepdims=True)
        e = jnp.exp(h2agg - cmax)
        out_ref[...] = e / jnp.sum(e, axis=1, keepdims=True)

    idx = (i > 0).astype(jnp.int32) + (i > nblk).astype(jnp.int32)
    jax.lax.switch(idx, [setup, phase_a, phase_b])


@jax.jit
def kernel(x, a, layout, kernel1, attn_s1, attn_n1, bias1, Wl, bl,
           kernel2, attn_s2, attn_n2, bias2):
    N, F = x.shape
    H, C = attn_s1.shape
    NC = attn_s2.shape[1]
    HC = H * C
    DL = layout.shape[1]
    nblk = N // BN

    def a_map(i):
        return (jnp.where(i <= nblk, jnp.maximum(i - 1, 0), i - nblk - 1), 0)

    out = pl.pallas_call(
        functools.partial(_body, heads=H, chan=C, nc=NC, bn=BN, nblk=nblk),
        grid=(1 + 2 * nblk,),
        in_specs=[
            pl.BlockSpec((N, F), lambda i: (0, 0)),
            pl.BlockSpec((BN, N), a_map),
            pl.BlockSpec((N, DL), lambda i: (0, 0)),
            pl.BlockSpec((F, H, C), lambda i: (0, 0, 0)),
            pl.BlockSpec((H, C), lambda i: (0, 0)),
            pl.BlockSpec((H, C), lambda i: (0, 0)),
            pl.BlockSpec((HC,), lambda i: (0,)),
            pl.BlockSpec((DL, HC), lambda i: (0, 0)),
            pl.BlockSpec((HC,), lambda i: (0,)),
            pl.BlockSpec((HC, 1, NC), lambda i: (0, 0, 0)),
            pl.BlockSpec((1, NC), lambda i: (0, 0)),
            pl.BlockSpec((1, NC), lambda i: (0, 0)),
            pl.BlockSpec((NC,), lambda i: (0,)),
        ],
        out_specs=pl.BlockSpec(
            (BN, NC), lambda i: (jnp.maximum(i - nblk - 1, 0), 0)),
        out_shape=jax.ShapeDtypeStruct((N, NC), jnp.float32),
        scratch_shapes=[
            pltpu.VMEM((N, N), jnp.bfloat16),
            pltpu.VMEM((N, NCA), jnp.bfloat16),
            pltpu.VMEM((N, H * CA), jnp.bfloat16),
            pltpu.VMEM((N, H), jnp.float32),
            pltpu.VMEM((N, H), jnp.float32),
            pltpu.VMEM((H, N), jnp.float32),
            pltpu.VMEM((H, N), jnp.float32),
            pltpu.VMEM((N, HC), jnp.float32),
            pltpu.VMEM((N, 1), jnp.float32),
            pltpu.VMEM((1, N), jnp.float32),
        ],
    )(x, a, layout, kernel1, attn_s1, attn_n1, bias1,
      Wl, bl, kernel2, attn_s2, attn_n2, bias2)
    return out
